# baseline (speedup 1.0000x reference)
import functools

import jax
import jax.numpy as jnp
from jax import lax
from jax.experimental import pallas as pl
from jax.experimental.pallas import tpu as pltpu
from jax.experimental.pallas import tpu_sc as plsc

K = 8192
DIM = 256
MEM_DIM = 1024
Q_DIM = 1024
B = 32
S = 2048

_BF = jnp.bfloat16
_F32 = jnp.float32


def _bdot(a, b, dims):
    return lax.dot_general(a.astype(_BF), b.astype(_BF), (dims, ((), ())),
                           preferred_element_type=_F32)


def _attn_vq_body(mem_ref, lens_ref, q_ref, ws_ref, wo_ref, emb_ref,
                  d_ref, v_ref):
    b = pl.program_id(0)
    mem = mem_ref[0]
    mem_bf = mem.astype(_BF)
    q = q_ref[0]
    qp = _bdot(q, ws_ref[...], (((1,), (1,))))
    scores = lax.dot_general(qp.astype(_BF), mem_bf, (((1,), (1,)), ((), ())),
                             preferred_element_type=_F32)
    pos = lax.broadcasted_iota(jnp.int32, (1, S), 1)
    scores = jnp.where(pos < lens_ref[b], scores, -1e9)
    m = jnp.max(scores)
    e = jnp.exp(scores - m)
    alpha = e / jnp.sum(e)
    context = lax.dot_general(alpha.astype(_BF), mem_bf,
                              (((1,), (0,)), ((), ())),
                              preferred_element_type=_F32)
    cat = jnp.concatenate([context, q], axis=1)
    V = jnp.tanh(_bdot(cat, wo_ref[...], (((1,), (0,)))))
    v_ref[...] = V.reshape(1, 1, DIM)
    diff = V.reshape(1, DIM) - emb_ref[...]
    d_ref[...] = jnp.sum(diff * diff, axis=1).reshape(1, 1, K)


def _attn_vq(input_memory, input_lens, init_query, W_score, W_out, emb):
    return pl.pallas_call(
        _attn_vq_body,
        grid=(B,),
        in_specs=[
            pl.BlockSpec((1, S, MEM_DIM), lambda b: (b, 0, 0)),
            pl.BlockSpec(memory_space=pltpu.SMEM),
            pl.BlockSpec((1, 1, Q_DIM), lambda b: (b, 0, 0)),
            pl.BlockSpec((MEM_DIM, Q_DIM), lambda b: (0, 0)),
            pl.BlockSpec((MEM_DIM + Q_DIM, DIM), lambda b: (0, 0)),
            pl.BlockSpec((K, DIM), lambda b: (0, 0)),
        ],
        out_specs=[
            pl.BlockSpec((1, 1, K), lambda b: (b, 0, 0)),
            pl.BlockSpec((1, 1, DIM), lambda b: (b, 0, 0)),
        ],
        out_shape=[
            jax.ShapeDtypeStruct((B, 1, K), _F32),
            jax.ShapeDtypeStruct((B, 1, DIM), _F32),
        ],
    )(input_memory, input_lens, init_query.reshape(B, 1, Q_DIM),
      W_score, W_out, emb)


def _lane_perm(x, perm):
    return lax.gather(
        x, perm.reshape(16, 1),
        lax.GatherDimensionNumbers(offset_dims=(), collapsed_slice_dims=(0,),
                                   start_index_map=(0,)),
        (1,), mode=lax.GatherScatterMode.PROMISE_IN_BOUNDS)


def _lane_reduce(x, op):
    lane = lax.iota(jnp.int32, 16)
    for k in (1, 2, 4, 8):
        x = op(x, _lane_perm(x, lane ^ k))
    return x


def _sc_body(d_hbm, emb_hbm, v_hbm, coll_hbm, diff_hbm,
             d_v, wq_v, v_v, out_v, sem):
    info = plsc.get_sparse_core_info()
    nc = info.num_cores
    wid = lax.axis_index("s") * nc + lax.axis_index("c")

    pltpu.sync_copy(d_hbm.at[wid], d_v)
    pltpu.sync_copy(v_hbm.at[wid], v_v)

    lane = lax.iota(jnp.int32, 16)
    big = jnp.float32(3.4e38)

    def amin_step(j, carry):
        vmin, imin = carry
        chunk = d_v[pl.ds(j * 16, 16)]
        upd = chunk < vmin
        return (jnp.where(upd, chunk, vmin),
                jnp.where(upd, lane + j * 16, imin))

    vmin0 = jnp.full((16,), big, _F32)
    imin0 = jnp.full((16,), 2**30, jnp.int32)
    vmin, imin = lax.fori_loop(0, K // 16, amin_step, (vmin0, imin0))
    gmin = _lane_reduce(vmin, jnp.minimum)
    gidx = _lane_reduce(jnp.where(vmin == gmin, imin, 2**30), jnp.minimum)

    pltpu.async_copy(emb_hbm.at[gidx], wq_v, sem).wait()

    acc = jnp.zeros((16,), _F32)
    for c in range(DIM // 16):
        t = wq_v[0, pl.ds(c * 16, 16)] - v_v[pl.ds(c * 16, 16)]
        acc = acc + t * t
    diff = _lane_reduce(acc, jnp.add)

    pltpu.sync_copy(wq_v.at[0], coll_hbm.at[wid, 0])
    out_v[...] = diff
    pltpu.sync_copy(out_v, diff_hbm.at[wid])


def _sc_select(d, emb, V):
    mesh = plsc.VectorSubcoreMesh(core_axis_name="c", subcore_axis_name="s")
    k = pl.kernel(
        _sc_body,
        out_type=(jax.ShapeDtypeStruct((B, 1, DIM), _F32),
                  jax.ShapeDtypeStruct((B, 16), _F32)),
        mesh=mesh,
        scratch_types=[
            pltpu.VMEM((K,), _F32),
            pltpu.VMEM((16, DIM), _F32),
            pltpu.VMEM((DIM,), _F32),
            pltpu.VMEM((16,), _F32),
            pltpu.SemaphoreType.DMA,
        ],
    )
    return k(d, emb, V)


def kernel(input_memory, input_lens, init_query, W_score, W_out, emb):
    lens32 = input_lens.astype(jnp.int32)
    d3, V3 = _attn_vq(input_memory, lens32, init_query, W_score, W_out, emb)
    collected, diffpad = _sc_select(d3.reshape(B, K), emb, V3.reshape(B, DIM))
    diff = diffpad[:, 0]
    return (collected, diff, diff)
